# baseline (speedup 1.0000x reference)
import jax
import jax.numpy as jnp
from jax.experimental import pallas as pl
from jax.experimental.pallas import tpu as pltpu

_BK = 1024


def _vq_kernel(vseg_ref, cdbl_ref,
               a0_ref, a1_ref, a2_ref, l0_ref, l1_ref, l2_ref,
               runval_ref, runchunk_ref, vsq_ref):
    j = pl.program_id(0)
    m = vseg_ref.shape[1]
    v = vseg_ref[0]
    cdbl = cdbl_ref[...]
    cross2 = jax.lax.dot_general(
        v, cdbl, (((1,), (1,)), ((), ())),
        preferred_element_type=jnp.float32)
    c2 = jnp.sum(cdbl * cdbl, axis=1) * 0.25

    @pl.when(j <= 2)
    def _init():
        v2 = jnp.sum(v * v, axis=1)
        vsq_ref[...] = jnp.broadcast_to(v2[:, None], (m, 128))
        runval_ref[...] = jnp.full((m, 128), jnp.inf, jnp.float32)
        runchunk_ref[...] = jnp.zeros((m, 128), jnp.int32)

    chunk_base = jnp.where(j <= 1, 0, (j - 2) * 8)
    nrb = 2
    rb = m // nrb
    for r in range(nrb):
        rsl = slice(r * rb, (r + 1) * rb)
        v2b = vsq_ref[rsl, :]
        rv = runval_ref[rsl, :]
        rc = runchunk_ref[rsl, :]
        for t in range(_BK // 128):
            sl = slice(t * 128, (t + 1) * 128)
            dist = (v2b + c2[None, sl]) - cross2[rsl, sl]
            better = dist < rv
            rv = jnp.minimum(rv, dist)
            rc = jnp.where(better, jnp.int32(chunk_base + t), rc)
        runval_ref[rsl, :] = rv
        runchunk_ref[rsl, :] = rc

    def _finalize(argmin_ref, losssum_ref):
        rv = runval_ref[...]
        rc = runchunk_ref[...]
        gmin = jnp.min(rv, axis=1)
        lane = jax.lax.broadcasted_iota(jnp.int32, (m, 128), 1)
        kidx = rc * 128 + lane
        cand = jnp.where(rv == gmin[:, None], kidx, 2147483647)
        argmin_ref[...] = jnp.min(cand, axis=1)
        losssum_ref[0, 0] = jnp.sum(gmin)

    @pl.when(j == 0)
    def _fin0():
        _finalize(a0_ref, l0_ref)

    @pl.when(j == 1)
    def _fin1():
        _finalize(a1_ref, l1_ref)

    @pl.when(j == 9)
    def _fin2():
        _finalize(a2_ref, l2_ref)


def kernel(v0, v1, v2, c0, c1, c2):
    b, s, d = v0.shape
    m = b * s
    pad = jnp.full((_BK - c0.shape[0], d), 1e6, jnp.float32)
    ccat = jnp.concatenate([c0, pad, c1, c2], axis=0)
    vcat = jnp.stack([v0.reshape(m, d), v1.reshape(m, d), v2.reshape(m, d)])
    nk = ccat.shape[0] // _BK
    a0, a1, a2, l0, l1, l2 = pl.pallas_call(
        _vq_kernel,
        grid=(nk,),
        in_specs=[
            pl.BlockSpec((1, m, d), lambda j: (jnp.minimum(j, 2), 0, 0)),
            pl.BlockSpec((_BK, d), lambda j: (j, 0)),
        ],
        out_specs=[
            pl.BlockSpec((m,), lambda j: (0,)),
            pl.BlockSpec((m,), lambda j: (0,)),
            pl.BlockSpec((m,), lambda j: (0,)),
            pl.BlockSpec(memory_space=pltpu.SMEM),
            pl.BlockSpec(memory_space=pltpu.SMEM),
            pl.BlockSpec(memory_space=pltpu.SMEM),
        ],
        out_shape=[
            jax.ShapeDtypeStruct((m,), jnp.int32),
            jax.ShapeDtypeStruct((m,), jnp.int32),
            jax.ShapeDtypeStruct((m,), jnp.int32),
            jax.ShapeDtypeStruct((1, 1), jnp.float32),
            jax.ShapeDtypeStruct((1, 1), jnp.float32),
            jax.ShapeDtypeStruct((1, 1), jnp.float32),
        ],
        scratch_shapes=[
            pltpu.VMEM((m, 128), jnp.float32),
            pltpu.VMEM((m, 128), jnp.int32),
            pltpu.VMEM((m, 128), jnp.float32),
        ],
        compiler_params=pltpu.CompilerParams(
            dimension_semantics=("arbitrary",)),
    )(vcat, ccat + ccat)
    losses = jnp.stack([l0[0, 0], l1[0, 0], l2[0, 0]]) / jnp.float32(m)
    loss = jnp.mean(losses)
    return (loss, a0.reshape(b, s), a1.reshape(b, s), a2.reshape(b, s))


# speedup vs baseline: 1.1270x; 1.1270x over previous
import functools

import jax
import jax.numpy as jnp
from jax.experimental import pallas as pl
from jax.experimental.pallas import tpu as pltpu


def _vq_tile_kernel(v_ref, cdbl_ref, argmin_ref, losssum_ref,
                    runval_ref, runchunk_ref, v2_ref, *, bk, nk):
    j = pl.program_id(0)
    m = v_ref.shape[0]
    v = v_ref[...]
    cdbl = cdbl_ref[...]
    cross2 = jax.lax.dot_general(
        v, cdbl, (((1,), (1,)), ((), ())),
        preferred_element_type=jnp.float32)
    c2 = jnp.sum(cdbl * cdbl, axis=1) * 0.25

    @pl.when(j == 0)
    def _init():
        v2 = jnp.sum(v * v, axis=1)
        v2_ref[...] = jnp.broadcast_to(v2[:, None], (m, 128))
        runval_ref[...] = jnp.full((m, 128), jnp.inf, jnp.float32)
        runchunk_ref[...] = jnp.zeros((m, 128), jnp.int32)

    v2b = v2_ref[...]
    rv = runval_ref[...]
    rc = runchunk_ref[...]
    for t in range(bk // 128):
        sl = slice(t * 128, (t + 1) * 128)
        dist = (v2b + c2[None, sl]) - cross2[:, sl]
        better = dist < rv
        rv = jnp.minimum(rv, dist)
        chunkno = j * (bk // 128) + t
        rc = jnp.where(better, jnp.int32(chunkno), rc)
    runval_ref[...] = rv
    runchunk_ref[...] = rc

    @pl.when(j == nk - 1)
    def _finalize():
        gmin = jnp.min(rv, axis=1)
        lane = jax.lax.broadcasted_iota(jnp.int32, (m, 128), 1)
        kidx = rc * 128 + lane
        cand = jnp.where(rv == gmin[:, None], kidx, 2147483647)
        argmin_ref[...] = jnp.min(cand, axis=1)
        losssum_ref[0, 0] = jnp.sum(gmin)


def _vq_assign(v2d, c, bk):
    m, d = v2d.shape
    k = c.shape[0]
    bk = min(bk, k)
    nk = k // bk
    argmin, losssum = pl.pallas_call(
        functools.partial(_vq_tile_kernel, bk=bk, nk=nk),
        grid=(nk,),
        in_specs=[
            pl.BlockSpec((m, d), lambda j: (0, 0)),
            pl.BlockSpec((bk, d), lambda j: (j, 0)),
        ],
        out_specs=[
            pl.BlockSpec((m,), lambda j: (0,)),
            pl.BlockSpec(memory_space=pltpu.SMEM),
        ],
        out_shape=[
            jax.ShapeDtypeStruct((m,), jnp.int32),
            jax.ShapeDtypeStruct((1, 1), jnp.float32),
        ],
        scratch_shapes=[
            pltpu.VMEM((m, 128), jnp.float32),
            pltpu.VMEM((m, 128), jnp.int32),
            pltpu.VMEM((m, 128), jnp.float32),
        ],
        compiler_params=pltpu.CompilerParams(
            dimension_semantics=("arbitrary",)),
    )(v2d, c + c)
    return argmin, losssum[0, 0]


def kernel(v0, v1, v2, c0, c1, c2):
    b, s, d = v0.shape
    m = b * s
    outs = []
    for v, c in ((v0, c0), (v1, c1), (v2, c2)):
        outs.append(_vq_assign(v.reshape(m, d), c, bk=1024))
    losses = jnp.stack([o[1] for o in outs]) / jnp.float32(m)
    loss = jnp.mean(losses)
    a0, a1, a2 = (o[0].reshape(b, s) for o in outs)
    return (loss, a0, a1, a2)


# speedup vs baseline: 1.2660x; 1.1234x over previous
import functools

import jax
import jax.numpy as jnp
from jax.experimental import pallas as pl
from jax.experimental.pallas import tpu as pltpu


def _vq_tile_kernel(v_ref, c_ref, argmin_ref, losssum_ref,
                    runval_ref, runchunk_ref, v2_ref, *, bk, nk):
    j = pl.program_id(0)
    m = v_ref.shape[0]
    v = v_ref[...]
    c = c_ref[...]
    cross2 = jax.lax.dot_general(
        v, c + c, (((1,), (1,)), ((), ())),
        preferred_element_type=jnp.float32)
    c2 = jnp.sum(c * c, axis=1)

    @pl.when(j == 0)
    def _init():
        v2 = jnp.sum(v * v, axis=1)
        v2_ref[...] = jnp.broadcast_to(v2[:, None], (m, 128))
        runval_ref[...] = jnp.full((m, 128), jnp.inf, jnp.float32)
        runchunk_ref[...] = jnp.zeros((m, 128), jnp.int32)

    nrb = 2
    rb = m // nrb
    for r in range(nrb):
        rsl = slice(r * rb, (r + 1) * rb)
        v2b = v2_ref[rsl, :]
        rv = runval_ref[rsl, :]
        rc = runchunk_ref[rsl, :]
        for t in range(bk // 128):
            sl = slice(t * 128, (t + 1) * 128)
            dist = (v2b + c2[None, sl]) - cross2[rsl, sl]
            better = dist < rv
            rv = jnp.minimum(rv, dist)
            chunkno = j * (bk // 128) + t
            rc = jnp.where(better, jnp.int32(chunkno), rc)
        runval_ref[rsl, :] = rv
        runchunk_ref[rsl, :] = rc

    @pl.when(j == nk - 1)
    def _finalize():
        rv = runval_ref[...]
        rc = runchunk_ref[...]
        gmin = jnp.min(rv, axis=1)
        lane = jax.lax.broadcasted_iota(jnp.int32, (m, 128), 1)
        kidx = rc * 128 + lane
        cand = jnp.where(rv == gmin[:, None], kidx, 2147483647)
        argmin_ref[...] = jnp.min(cand, axis=1)
        losssum_ref[0, 0] = jnp.sum(gmin)


def _vq_assign(v2d, c, bk):
    m, d = v2d.shape
    k = c.shape[0]
    bk = min(bk, k)
    nk = k // bk
    argmin, losssum = pl.pallas_call(
        functools.partial(_vq_tile_kernel, bk=bk, nk=nk),
        grid=(nk,),
        in_specs=[
            pl.BlockSpec((m, d), lambda j: (0, 0)),
            pl.BlockSpec((bk, d), lambda j: (j, 0)),
        ],
        out_specs=[
            pl.BlockSpec((m,), lambda j: (0,)),
            pl.BlockSpec(memory_space=pltpu.SMEM),
        ],
        out_shape=[
            jax.ShapeDtypeStruct((m,), jnp.int32),
            jax.ShapeDtypeStruct((1, 1), jnp.float32),
        ],
        scratch_shapes=[
            pltpu.VMEM((m, 128), jnp.float32),
            pltpu.VMEM((m, 128), jnp.int32),
            pltpu.VMEM((m, 128), jnp.float32),
        ],
        compiler_params=pltpu.CompilerParams(
            dimension_semantics=("arbitrary",)),
    )(v2d, c)
    return argmin, losssum[0, 0]


def kernel(v0, v1, v2, c0, c1, c2):
    b, s, d = v0.shape
    m = b * s
    outs = []
    for v, c in ((v0, c0), (v1, c1), (v2, c2)):
        outs.append(_vq_assign(v.reshape(m, d), c, bk=1024))
    losses = jnp.stack([o[1] for o in outs]) / jnp.float32(m)
    loss = jnp.mean(losses)
    a0, a1, a2 = (o[0].reshape(b, s) for o in outs)
    return (loss, a0, a1, a2)


# speedup vs baseline: 1.2688x; 1.0022x over previous
import functools

import jax
import jax.numpy as jnp
from jax.experimental import pallas as pl
from jax.experimental.pallas import tpu as pltpu


def _vq_tile_kernel(v_ref, c_ref, argmin_ref, losssum_ref,
                    runval_ref, runchunk_ref, v2_ref, *, bk, nk):
    j = pl.program_id(0)
    m = v_ref.shape[0]
    v = v_ref[...]
    c = c_ref[...]
    cross2 = jax.lax.dot_general(
        v, c + c, (((1,), (1,)), ((), ())),
        preferred_element_type=jnp.float32)
    c2 = jnp.sum(c * c, axis=1)

    @pl.when(j == 0)
    def _init():
        v2 = jnp.sum(v * v, axis=1)
        v2_ref[...] = jnp.broadcast_to(v2[:, None], (m, 128))
        runval_ref[...] = jnp.full((m, 128), jnp.inf, jnp.float32)
        runchunk_ref[...] = jnp.zeros((m, 128), jnp.int32)

    nrb = 4
    rb = m // nrb
    for r in range(nrb):
        rsl = slice(r * rb, (r + 1) * rb)
        v2b = v2_ref[rsl, :]
        rv = runval_ref[rsl, :]
        rc = runchunk_ref[rsl, :]
        for t in range(bk // 128):
            sl = slice(t * 128, (t + 1) * 128)
            dist = (v2b + c2[None, sl]) - cross2[rsl, sl]
            better = dist < rv
            rv = jnp.minimum(rv, dist)
            chunkno = j * (bk // 128) + t
            rc = jnp.where(better, jnp.int32(chunkno), rc)
        runval_ref[rsl, :] = rv
        runchunk_ref[rsl, :] = rc

    @pl.when(j == nk - 1)
    def _finalize():
        rv = runval_ref[...]
        rc = runchunk_ref[...]
        gmin = jnp.min(rv, axis=1)
        lane = jax.lax.broadcasted_iota(jnp.int32, (m, 128), 1)
        kidx = rc * 128 + lane
        cand = jnp.where(rv == gmin[:, None], kidx, 2147483647)
        argmin_ref[...] = jnp.min(cand, axis=1)
        losssum_ref[0, 0] = jnp.sum(gmin)


def _vq_assign(v2d, c, bk):
    m, d = v2d.shape
    k = c.shape[0]
    bk = min(bk, k)
    nk = k // bk
    argmin, losssum = pl.pallas_call(
        functools.partial(_vq_tile_kernel, bk=bk, nk=nk),
        grid=(nk,),
        in_specs=[
            pl.BlockSpec((m, d), lambda j: (0, 0)),
            pl.BlockSpec((bk, d), lambda j: (j, 0)),
        ],
        out_specs=[
            pl.BlockSpec((m,), lambda j: (0,)),
            pl.BlockSpec(memory_space=pltpu.SMEM),
        ],
        out_shape=[
            jax.ShapeDtypeStruct((m,), jnp.int32),
            jax.ShapeDtypeStruct((1, 1), jnp.float32),
        ],
        scratch_shapes=[
            pltpu.VMEM((m, 128), jnp.float32),
            pltpu.VMEM((m, 128), jnp.int32),
            pltpu.VMEM((m, 128), jnp.float32),
        ],
        compiler_params=pltpu.CompilerParams(
            dimension_semantics=("arbitrary",)),
    )(v2d, c)
    return argmin, losssum[0, 0]


def kernel(v0, v1, v2, c0, c1, c2):
    b, s, d = v0.shape
    m = b * s
    outs = []
    for v, c in ((v0, c0), (v1, c1), (v2, c2)):
        outs.append(_vq_assign(v.reshape(m, d), c, bk=1024))
    losses = jnp.stack([o[1] for o in outs]) / jnp.float32(m)
    loss = jnp.mean(losses)
    a0, a1, a2 = (o[0].reshape(b, s) for o in outs)
    return (loss, a0, a1, a2)
